# baseline (speedup 1.0000x reference)
import functools

import jax
import jax.numpy as jnp
from jax import lax
from jax.experimental import pallas as pl
from jax.experimental.pallas import tpu as pltpu
from jax.experimental.pallas import tpu_sc as plsc

N = 10000
D = 128
E = 320000
NPAD = 10240
NW = 32
B = 128
CHUNKS = 79
EPT = CHUNKS * B
EPAD = EPT * NW
RPT = NPAD // 16
RB = 512
GRID = NPAD // RB

_mesh = plsc.VectorSubcoreMesh(core_axis_name="c", subcore_axis_name="s")


@functools.partial(
    pl.kernel,
    mesh=_mesh,
    out_type=jax.ShapeDtypeStruct((2 * NPAD, D), jnp.float32),
    scratch_types=[
        pltpu.VMEM((CHUNKS, B), jnp.int32),
        pltpu.VMEM((B, D), jnp.float32),
        pltpu.VMEM_SHARED((NPAD, D), jnp.float32),
    ],
)
def _deg_kernel(dst_hbm, ones_hbm, zeros_hbm, out_hbm, dst_v, ones_v, acc):
    cid = lax.axis_index("c")
    sid = lax.axis_index("s")
    wid = sid * 2 + cid
    pltpu.sync_copy(zeros_hbm, acc.at[pl.ds(sid * RPT, RPT)])
    pltpu.sync_copy(ones_hbm, ones_v)
    pltpu.sync_copy(dst_hbm.at[wid], dst_v)
    plsc.subcore_barrier()

    def body(j, c):
        pltpu.sync_copy(ones_v, acc.at[dst_v.at[j]], add=True)
        return c

    lax.fori_loop(0, CHUNKS, body, 0)
    plsc.subcore_barrier()
    pltpu.sync_copy(acc.at[pl.ds(sid * RPT, RPT)],
                    out_hbm.at[pl.ds(cid * NPAD + sid * RPT, RPT)])


@functools.partial(
    pl.kernel,
    mesh=_mesh,
    out_type=jax.ShapeDtypeStruct((2 * NPAD, D), jnp.float32),
    scratch_types=[
        pltpu.VMEM((CHUNKS, B), jnp.int32),
        pltpu.VMEM((CHUNKS, B), jnp.int32),
        pltpu.VMEM((B, D), jnp.float32),
        pltpu.VMEM_SHARED((NPAD, D), jnp.float32),
        pltpu.SemaphoreType.DMA,
    ],
)
def _scatter_kernel(y_hbm, src_hbm, dst_hbm, zeros_hbm, out_hbm,
                    src_v, dst_v, rows_v, acc, sem):
    cid = lax.axis_index("c")
    sid = lax.axis_index("s")
    wid = sid * 2 + cid
    pltpu.sync_copy(zeros_hbm, acc.at[pl.ds(sid * RPT, RPT)])
    pltpu.sync_copy(src_hbm.at[wid], src_v)
    pltpu.sync_copy(dst_hbm.at[wid], dst_v)
    plsc.subcore_barrier()

    def body(j, c):
        pltpu.async_copy(y_hbm.at[src_v.at[j]], rows_v, sem).wait()
        pltpu.sync_copy(rows_v, acc.at[dst_v.at[j]], add=True)
        return c

    lax.fori_loop(0, CHUNKS, body, 0)
    plsc.subcore_barrier()
    pltpu.sync_copy(acc.at[pl.ds(sid * RPT, RPT)],
                    out_hbm.at[pl.ds(cid * NPAD + sid * RPT, RPT)])


def _t1_body(x_ref, w_ref, degp_ref, y_ref, disb_ref):
    c = degp_ref[0, :, 0:1] + degp_ref[1, :, 0:1] + 1.0
    disb = lax.rsqrt(jnp.broadcast_to(c, (RB, D)))
    disb_ref[...] = disb
    y_ref[...] = jnp.dot(x_ref[...], w_ref[...],
                         preferred_element_type=jnp.float32) * disb


_t1 = pl.pallas_call(
    _t1_body,
    grid=(GRID,),
    in_specs=[
        pl.BlockSpec((RB, D), lambda i: (i, 0)),
        pl.BlockSpec((D, D), lambda i: (0, 0)),
        pl.BlockSpec((2, RB, D), lambda i: (0, i, 0)),
    ],
    out_specs=[
        pl.BlockSpec((RB, D), lambda i: (i, 0)),
        pl.BlockSpec((RB, D), lambda i: (i, 0)),
    ],
    out_shape=[
        jax.ShapeDtypeStruct((NPAD, D), jnp.float32),
        jax.ShapeDtypeStruct((NPAD, D), jnp.float32),
    ],
)


def _tmid_body(s_ref, y_ref, disb_ref, b_ref, w_ref, o_ref):
    disb = disb_ref[...]
    h = jnp.maximum((s_ref[0] + s_ref[1] + y_ref[...]) * disb + b_ref[...],
                    0.0)
    o_ref[...] = jnp.dot(h, w_ref[...],
                         preferred_element_type=jnp.float32) * disb


_tmid = pl.pallas_call(
    _tmid_body,
    grid=(GRID,),
    in_specs=[
        pl.BlockSpec((2, RB, D), lambda i: (0, i, 0)),
        pl.BlockSpec((RB, D), lambda i: (i, 0)),
        pl.BlockSpec((RB, D), lambda i: (i, 0)),
        pl.BlockSpec((1, D), lambda i: (0, 0)),
        pl.BlockSpec((D, D), lambda i: (0, 0)),
    ],
    out_specs=pl.BlockSpec((RB, D), lambda i: (i, 0)),
    out_shape=jax.ShapeDtypeStruct((NPAD, D), jnp.float32),
)


def _tfin_body(s_ref, y_ref, disb_ref, b_ref, o_ref):
    o_ref[...] = ((s_ref[0] + s_ref[1] + y_ref[...]) * disb_ref[...]
                  + b_ref[...])


_tfin = pl.pallas_call(
    _tfin_body,
    grid=(GRID,),
    in_specs=[
        pl.BlockSpec((2, RB, D), lambda i: (0, i, 0)),
        pl.BlockSpec((RB, D), lambda i: (i, 0)),
        pl.BlockSpec((RB, D), lambda i: (i, 0)),
        pl.BlockSpec((1, D), lambda i: (0, 0)),
    ],
    out_specs=pl.BlockSpec((RB, D), lambda i: (i, 0)),
    out_shape=jax.ShapeDtypeStruct((NPAD, D), jnp.float32),
)


def kernel(x, edge_index, W1, b1, W2, b2, W3, b3):
    src = edge_index[0].astype(jnp.int32)
    dst = edge_index[1].astype(jnp.int32)
    pad_e = EPAD - E
    src_p = jnp.concatenate(
        [src, jnp.zeros((pad_e,), jnp.int32)]).reshape(NW, CHUNKS, B)
    dst_p = jnp.concatenate(
        [dst, jnp.full((pad_e,), NPAD - 1, jnp.int32)]).reshape(NW, CHUNKS, B)
    x_p = jnp.pad(x, ((0, NPAD - N), (0, 0)))
    zD = jnp.zeros((RPT, D), jnp.float32)
    onesD = jnp.ones((B, D), jnp.float32)
    b1r = b1.reshape(1, D)
    b2r = b2.reshape(1, D)
    b3r = b3.reshape(1, D)

    degp = _deg_kernel(dst_p, onesD, zD).reshape(2, NPAD, D)
    y1, disb = _t1(x_p, W1, degp)
    s1 = _scatter_kernel(y1, src_p, dst_p, zD).reshape(2, NPAD, D)
    y2 = _tmid(s1, y1, disb, b1r, W2)
    s2 = _scatter_kernel(y2, src_p, dst_p, zD).reshape(2, NPAD, D)
    y3 = _tmid(s2, y2, disb, b2r, W3)
    s3 = _scatter_kernel(y3, src_p, dst_p, zD).reshape(2, NPAD, D)
    out = _tfin(s3, y3, disb, b3r)
    return out[:N]


# speedup vs baseline: 1.0004x; 1.0004x over previous
import functools

import jax
import jax.numpy as jnp
from jax import lax
from jax.experimental import pallas as pl
from jax.experimental.pallas import tpu as pltpu
from jax.experimental.pallas import tpu_sc as plsc

N = 10000
D = 128
E = 320000
NPAD = 10240
NW = 32
B = 128
CHUNKS = 79
EPT = CHUNKS * B
EPAD = EPT * NW
RPT = NPAD // 16
RB = 512
GRID = NPAD // RB

_mesh = plsc.VectorSubcoreMesh(core_axis_name="c", subcore_axis_name="s")


@functools.partial(
    pl.kernel,
    mesh=_mesh,
    out_type=jax.ShapeDtypeStruct((2 * NPAD, D), jnp.float32),
    scratch_types=[
        pltpu.VMEM((CHUNKS, B), jnp.int32),
        pltpu.VMEM((B, D), jnp.float32),
        pltpu.VMEM_SHARED((NPAD, D), jnp.float32),
        pltpu.SemaphoreType.DMA,
    ],
)
def _deg_kernel(dst_hbm, ones_hbm, zeros_hbm, out_hbm, dst_v, ones_v, acc,
                sem):
    cid = lax.axis_index("c")
    sid = lax.axis_index("s")
    wid = sid * 2 + cid
    pltpu.sync_copy(zeros_hbm, acc.at[pl.ds(sid * RPT, RPT)])
    pltpu.sync_copy(ones_hbm, ones_v)
    pltpu.sync_copy(dst_hbm.at[wid], dst_v)
    plsc.subcore_barrier()

    def body(j, c):
        pltpu.async_copy(ones_v, acc.at[dst_v.at[j]], sem, add=True)

        @pl.when(j >= 4)
        def _():
            pltpu.make_async_copy(ones_v, acc.at[dst_v.at[j]], sem).wait()
        return c

    lax.fori_loop(0, CHUNKS, body, 0)

    def drain(j, c):
        pltpu.make_async_copy(ones_v, acc.at[dst_v.at[j]], sem).wait()
        return c

    lax.fori_loop(0, 4, drain, 0)
    plsc.subcore_barrier()
    pltpu.sync_copy(acc.at[pl.ds(sid * RPT, RPT)],
                    out_hbm.at[pl.ds(cid * NPAD + sid * RPT, RPT)])


@functools.partial(
    pl.kernel,
    mesh=_mesh,
    out_type=jax.ShapeDtypeStruct((2 * NPAD, D), jnp.float32),
    scratch_types=[
        pltpu.VMEM((CHUNKS, B), jnp.int32),
        pltpu.VMEM((CHUNKS, B), jnp.int32),
        pltpu.VMEM((B, D), jnp.float32),
        pltpu.VMEM_SHARED((NPAD, D), jnp.float32),
        pltpu.SemaphoreType.DMA,
    ],
)
def _scatter_kernel(y_hbm, src_hbm, dst_hbm, zeros_hbm, out_hbm,
                    src_v, dst_v, rows_v, acc, sem):
    cid = lax.axis_index("c")
    sid = lax.axis_index("s")
    wid = sid * 2 + cid
    pltpu.sync_copy(zeros_hbm, acc.at[pl.ds(sid * RPT, RPT)])
    pltpu.sync_copy(src_hbm.at[wid], src_v)
    pltpu.sync_copy(dst_hbm.at[wid], dst_v)
    plsc.subcore_barrier()

    def body(j, c):
        pltpu.async_copy(y_hbm.at[src_v.at[j]], rows_v, sem).wait()
        pltpu.sync_copy(rows_v, acc.at[dst_v.at[j]], add=True)
        return c

    lax.fori_loop(0, CHUNKS, body, 0)
    plsc.subcore_barrier()
    pltpu.sync_copy(acc.at[pl.ds(sid * RPT, RPT)],
                    out_hbm.at[pl.ds(cid * NPAD + sid * RPT, RPT)])


def _t1_body(x_ref, w_ref, degp_ref, y_ref, disb_ref):
    c = degp_ref[0, :, 0:1] + degp_ref[1, :, 0:1] + 1.0
    disb = lax.rsqrt(jnp.broadcast_to(c, (RB, D)))
    disb_ref[...] = disb
    y_ref[...] = jnp.dot(x_ref[...], w_ref[...],
                         preferred_element_type=jnp.float32) * disb


_t1 = pl.pallas_call(
    _t1_body,
    grid=(GRID,),
    in_specs=[
        pl.BlockSpec((RB, D), lambda i: (i, 0)),
        pl.BlockSpec((D, D), lambda i: (0, 0)),
        pl.BlockSpec((2, RB, D), lambda i: (0, i, 0)),
    ],
    out_specs=[
        pl.BlockSpec((RB, D), lambda i: (i, 0)),
        pl.BlockSpec((RB, D), lambda i: (i, 0)),
    ],
    out_shape=[
        jax.ShapeDtypeStruct((NPAD, D), jnp.float32),
        jax.ShapeDtypeStruct((NPAD, D), jnp.float32),
    ],
)


def _tmid_body(s_ref, y_ref, disb_ref, b_ref, w_ref, o_ref):
    disb = disb_ref[...]
    h = jnp.maximum((s_ref[0] + s_ref[1] + y_ref[...]) * disb + b_ref[...],
                    0.0)
    o_ref[...] = jnp.dot(h, w_ref[...],
                         preferred_element_type=jnp.float32) * disb


_tmid = pl.pallas_call(
    _tmid_body,
    grid=(GRID,),
    in_specs=[
        pl.BlockSpec((2, RB, D), lambda i: (0, i, 0)),
        pl.BlockSpec((RB, D), lambda i: (i, 0)),
        pl.BlockSpec((RB, D), lambda i: (i, 0)),
        pl.BlockSpec((1, D), lambda i: (0, 0)),
        pl.BlockSpec((D, D), lambda i: (0, 0)),
    ],
    out_specs=pl.BlockSpec((RB, D), lambda i: (i, 0)),
    out_shape=jax.ShapeDtypeStruct((NPAD, D), jnp.float32),
)


def _tfin_body(s_ref, y_ref, disb_ref, b_ref, o_ref):
    o_ref[...] = ((s_ref[0] + s_ref[1] + y_ref[...]) * disb_ref[...]
                  + b_ref[...])


_tfin = pl.pallas_call(
    _tfin_body,
    grid=(GRID,),
    in_specs=[
        pl.BlockSpec((2, RB, D), lambda i: (0, i, 0)),
        pl.BlockSpec((RB, D), lambda i: (i, 0)),
        pl.BlockSpec((RB, D), lambda i: (i, 0)),
        pl.BlockSpec((1, D), lambda i: (0, 0)),
    ],
    out_specs=pl.BlockSpec((RB, D), lambda i: (i, 0)),
    out_shape=jax.ShapeDtypeStruct((NPAD, D), jnp.float32),
)


def kernel(x, edge_index, W1, b1, W2, b2, W3, b3):
    src = edge_index[0].astype(jnp.int32)
    dst = edge_index[1].astype(jnp.int32)
    pad_e = EPAD - E
    src_p = jnp.concatenate(
        [src, jnp.zeros((pad_e,), jnp.int32)]).reshape(NW, CHUNKS, B)
    dst_p = jnp.concatenate(
        [dst, jnp.full((pad_e,), NPAD - 1, jnp.int32)]).reshape(NW, CHUNKS, B)
    x_p = jnp.pad(x, ((0, NPAD - N), (0, 0)))
    zD = jnp.zeros((RPT, D), jnp.float32)
    onesD = jnp.ones((B, D), jnp.float32)
    b1r = b1.reshape(1, D)
    b2r = b2.reshape(1, D)
    b3r = b3.reshape(1, D)

    degp = _deg_kernel(dst_p, onesD, zD).reshape(2, NPAD, D)
    y1, disb = _t1(x_p, W1, degp)
    s1 = _scatter_kernel(y1, src_p, dst_p, zD).reshape(2, NPAD, D)
    y2 = _tmid(s1, y1, disb, b1r, W2)
    s2 = _scatter_kernel(y2, src_p, dst_p, zD).reshape(2, NPAD, D)
    y3 = _tmid(s2, y2, disb, b2r, W3)
    s3 = _scatter_kernel(y3, src_p, dst_p, zD).reshape(2, NPAD, D)
    out = _tfin(s3, y3, disb, b3r)
    return out[:N]
